# baseline (speedup 1.0000x reference)
import functools

import jax
import jax.numpy as jnp
from jax import lax
from jax.experimental import pallas as pl
from jax.experimental.pallas import tpu as pltpu
from jax.experimental.pallas import tpu_sc as plsc
from jax._src.random.threefry2x32 import threefry2x32_p

EXPRESSION_MASK_RATIO = 0.4
MIN_VISIBLE_GENES = 10

NB = 256
NW = 32
L = 16


def _sc_thresholds(rf, k1):
  R, G = rf.shape
  rpw = R // NW
  C = 8
  nch = rpw // C
  nvec = G // L
  nck = NB // L
  mesh = plsc.VectorSubcoreMesh(core_axis_name="c", subcore_axis_name="s")

  @functools.partial(
      pl.kernel,
      out_type=jax.ShapeDtypeStruct((R,), jnp.int32),
      mesh=mesh,
      compiler_params=pltpu.CompilerParams(needs_layout_passes=False),
      scratch_types=[
          pltpu.VMEM((2, C, G), jnp.float32),
          pltpu.VMEM((NB * L,), jnp.int32),
          pltpu.VMEM((G + L,), jnp.float32),
          pltpu.VMEM((rpw,), jnp.int32),
          pltpu.SemaphoreType.DMA,
          pltpu.SemaphoreType.DMA,
      ],
  )
  def body(x_hbm, out_hbm, buf, hist, surv, thr, sem0, sem1):
    wid = lax.axis_index("s") * 2 + lax.axis_index("c")
    base = wid * rpw
    lane = lax.iota(jnp.int32, L)
    ones = jnp.ones((L,), jnp.int32)
    zeros = jnp.zeros((L,), jnp.int32)
    fnb = jnp.float32(NB)

    @plsc.parallel_loop(0, NB, unroll=4)
    def _(b):
      hist[pl.ds(b * L, L)] = zeros

    def start(ch, slot, sem):
      pltpu.async_copy(x_hbm.at[pl.ds(base + ch * C, C)], buf.at[slot], sem)

    def wait(ch, slot, sem):
      pltpu.make_async_copy(
          x_hbm.at[pl.ds(base + ch * C, C)], buf.at[slot], sem).wait()

    def row_body(data, rr, row_slot):
      @plsc.parallel_loop(0, nvec, unroll=4)
      def _(i):
        v = data[rr, pl.ds(i * L, L)]
        idx = ((v * fnb).astype(jnp.int32) << 4) | lane
        plsc.addupdate_scatter(hist, [idx], ones)

      @plsc.parallel_loop(
          0, nck, unroll=2,
          carry=(jnp.int32(0), jnp.int32(0), jnp.int32(0), jnp.int32(0)))
      def scan_res(c, carry):
        cum, found, cstar, cumb = carry
        cbase = c * (L * L)
        acc = hist[pl.ds(cbase, L)]
        for j in range(1, L):
          acc = acc + hist[pl.ds(cbase + j * L, L)]
        newcum = cum + jnp.sum(acc)
        hit = (1 - found) * (newcum >= k1).astype(jnp.int32)
        cstar = jnp.where(hit == 1, c, cstar)
        cumb = jnp.where(hit == 1, cum, cumb)
        found = jnp.maximum(found, hit)
        return newcum, found, cstar, cumb
      _, _, cstar, cumb = scan_res

      kin = k1 - cumb
      def gb(i, carry):
        cum2, found2, bstar2, cumbb = carry
        cnt = jnp.sum(hist[pl.ds((cstar * L + i) * L, L)])
        newc = cum2 + cnt
        hit = (1 - found2) * (newc >= kin).astype(jnp.int32)
        bstar2 = jnp.where(hit == 1, cstar * L + i, bstar2)
        cumbb = jnp.where(hit == 1, cumb + cum2, cumbb)
        found2 = jnp.maximum(found2, hit)
        return newc, found2, bstar2, cumbb
      _, _, bstar, cum_below = lax.fori_loop(
          0, L, gb, (jnp.int32(0), jnp.int32(0), jnp.int32(0),
                     jnp.int32(0)))
      bstar_vec = jnp.full((L,), 1, jnp.int32) * bstar
      kpp = k1 - cum_below

      @plsc.parallel_loop(0, nvec, unroll=4, carry=jnp.int32(0))
      def s(i, off):
        v = data[rr, pl.ds(i * L, L)]
        bkt = (v * fnb).astype(jnp.int32)
        plsc.store_scatter(hist, [(bkt << 4) | lane], zeros)
        m = bkt == bstar_vec
        plsc.store_compressed(surv.at[pl.ds(off, L)], v, mask=m)
        return off + plsc.all_reduce_population_count(m)[0]

      surv[pl.ds(s, L)] = jnp.full((L,), jnp.inf, jnp.float32)
      nv = (s + L - 1) >> 4

      lo0 = lax.bitcast_convert_type(
          bstar.astype(jnp.float32) * jnp.float32(1.0 / NB), jnp.int32)
      hi0 = lax.bitcast_convert_type(
          (bstar + 1).astype(jnp.float32) * jnp.float32(1.0 / NB),
          jnp.int32) - 1

      def bs_cond(c):
        lo, hi = c
        return lo < hi

      def bs_step(c):
        lo, hi = c
        mid = lo + ((hi - lo) >> 1)

        def bcnt(i, a):
          vi = plsc.bitcast(surv[pl.ds(i * L, L)], jnp.int32)
          return a + jnp.sum((vi <= mid).astype(jnp.int32))
        cnt = lax.fori_loop(0, nv, bcnt, jnp.int32(0))

        ge = cnt >= kpp
        return jnp.where(ge, lo, mid + 1), jnp.where(ge, mid, hi)

      tpat, _ = lax.while_loop(bs_cond, bs_step, (lo0, hi0))
      plsc.store_scatter(
          thr, [jnp.full((L,), row_slot, jnp.int32)],
          jnp.full((L,), tpat, jnp.int32), mask=lane == 0)

    def chunk_pair(cp, _):
      ch0 = cp * 2
      wait(ch0, 0, sem0)
      start(ch0 + 1, 1, sem1)

      def rows_a(rr, _):
        row_body(buf.at[0], rr, ch0 * C + rr)
        return 0
      lax.fori_loop(0, C, rows_a, 0)

      wait(ch0 + 1, 1, sem1)

      @pl.when(ch0 + 2 < nch)
      def _():
        start(ch0 + 2, 0, sem0)

      def rows_b(rr, _):
        row_body(buf.at[1], rr, (ch0 + 1) * C + rr)
        return 0
      lax.fori_loop(0, C, rows_b, 0)
      return 0

    start(0, 0, sem0)
    lax.fori_loop(0, nch // 2, chunk_pair, 0)
    pltpu.sync_copy(thr, out_hbm.at[pl.ds(base, rpw)])

  return body(rf)


def _tc_mask(rf, thrf):
  R, G = rf.shape
  blk = 256

  def body(x_ref, t_ref, o_ref):
    o_ref[...] = x_ref[...] >= t_ref[...]

  return pl.pallas_call(
      body,
      grid=(R // blk,),
      in_specs=[
          pl.BlockSpec((blk, G), lambda i: (i, 0)),
          pl.BlockSpec((blk, 1), lambda i: (i, 0)),
      ],
      out_specs=pl.BlockSpec((blk, G), lambda i: (i, 0)),
      out_shape=jax.ShapeDtypeStruct((R, G), jnp.bool_),
  )(rf, thrf)


def _gen_slice(offset, m):
  i = lax.iota(jnp.uint32, m) + jnp.uint32(offset)
  z = jnp.zeros((m,), jnp.uint32)
  b1, b2 = threefry2x32_p.bind(jnp.uint32(0), jnp.uint32(42), z, i)
  fb = ((b1 ^ b2) >> 9) | jnp.uint32(0x3F800000)
  return lax.bitcast_convert_type(fb, jnp.float32) - jnp.float32(1.0)


@jax.jit
def kernel(expression):
  B, N, G = expression.shape
  num_masked = max(1, int(G * EXPRESSION_MASK_RATIO))
  num_masked = min(num_masked, G - MIN_VISIBLE_GENES)
  k1 = G - num_masked
  R = B * N
  Q = 8
  rpc = R // Q
  masks = []
  for q in range(Q):
    rfq = _gen_slice(q * rpc * G, rpc * G).reshape(rpc, G)
    thr_pat = _sc_thresholds(rfq, k1)
    thrf = lax.bitcast_convert_type(thr_pat, jnp.float32).reshape(-1, 1)
    masks.append(_tc_mask(rfq, thrf))
  return jnp.concatenate(masks, axis=0).reshape(B, N, G)


# speedup vs baseline: 1.0237x; 1.0237x over previous
import functools

import jax
import jax.numpy as jnp
from jax import lax
from jax.experimental import pallas as pl
from jax.experimental.pallas import tpu as pltpu
from jax.experimental.pallas import tpu_sc as plsc
from jax._src.random.threefry2x32 import threefry2x32_p

EXPRESSION_MASK_RATIO = 0.4
MIN_VISIBLE_GENES = 10

NB = 256
NW = 32
L = 16


def _sc_thresholds(rf, k1):
  R, G = rf.shape
  rpw = R // NW
  C = 8
  nch = rpw // C
  nvec = G // L
  nck = NB // L
  mesh = plsc.VectorSubcoreMesh(core_axis_name="c", subcore_axis_name="s")

  @functools.partial(
      pl.kernel,
      out_type=jax.ShapeDtypeStruct((R,), jnp.int32),
      mesh=mesh,
      compiler_params=pltpu.CompilerParams(needs_layout_passes=False),
      scratch_types=[
          pltpu.VMEM((2, C, G), jnp.float32),
          pltpu.VMEM((NB * L,), jnp.int32),
          pltpu.VMEM((G + L,), jnp.float32),
          pltpu.VMEM((rpw,), jnp.int32),
          pltpu.SemaphoreType.DMA,
          pltpu.SemaphoreType.DMA,
      ],
  )
  def body(x_hbm, out_hbm, buf, hist, surv, thr, sem0, sem1):
    wid = lax.axis_index("s") * 2 + lax.axis_index("c")
    base = wid * rpw
    lane = lax.iota(jnp.int32, L)
    ones = jnp.ones((L,), jnp.int32)
    zeros = jnp.zeros((L,), jnp.int32)
    fnb = jnp.float32(NB)

    @plsc.parallel_loop(0, NB, unroll=4)
    def _(b):
      hist[pl.ds(b * L, L)] = zeros

    def start(ch, slot, sem):
      pltpu.async_copy(x_hbm.at[pl.ds(base + ch * C, C)], buf.at[slot], sem)

    def wait(ch, slot, sem):
      pltpu.make_async_copy(
          x_hbm.at[pl.ds(base + ch * C, C)], buf.at[slot], sem).wait()

    def row_body(data, rr, row_slot):
      @plsc.parallel_loop(0, nvec, unroll=4)
      def _(i):
        v = data[rr, pl.ds(i * L, L)]
        idx = ((v * fnb).astype(jnp.int32) << 4) | lane
        plsc.addupdate_scatter(hist, [idx], ones)

      @plsc.parallel_loop(
          0, nck, unroll=2,
          carry=(jnp.int32(0), jnp.int32(0), jnp.int32(0), jnp.int32(0)))
      def scan_res(c, carry):
        cum, found, cstar, cumb = carry
        cbase = c * (L * L)
        acc = hist[pl.ds(cbase, L)]
        for j in range(1, L):
          acc = acc + hist[pl.ds(cbase + j * L, L)]
        newcum = cum + jnp.sum(acc)
        hit = (1 - found) * (newcum >= k1).astype(jnp.int32)
        cstar = jnp.where(hit == 1, c, cstar)
        cumb = jnp.where(hit == 1, cum, cumb)
        found = jnp.maximum(found, hit)
        return newcum, found, cstar, cumb
      _, _, cstar, cumb = scan_res

      kin = k1 - cumb
      def gb(i, carry):
        cum2, found2, bstar2, cumbb = carry
        cnt = jnp.sum(hist[pl.ds((cstar * L + i) * L, L)])
        newc = cum2 + cnt
        hit = (1 - found2) * (newc >= kin).astype(jnp.int32)
        bstar2 = jnp.where(hit == 1, cstar * L + i, bstar2)
        cumbb = jnp.where(hit == 1, cumb + cum2, cumbb)
        found2 = jnp.maximum(found2, hit)
        return newc, found2, bstar2, cumbb
      _, _, bstar, cum_below = lax.fori_loop(
          0, L, gb, (jnp.int32(0), jnp.int32(0), jnp.int32(0),
                     jnp.int32(0)))
      bstar_vec = jnp.full((L,), 1, jnp.int32) * bstar
      kpp = k1 - cum_below

      @plsc.parallel_loop(0, nvec, unroll=4, carry=jnp.int32(0))
      def s(i, off):
        v = data[rr, pl.ds(i * L, L)]
        bkt = (v * fnb).astype(jnp.int32)
        plsc.store_scatter(hist, [(bkt << 4) | lane], zeros)
        m = bkt == bstar_vec
        plsc.store_compressed(surv.at[pl.ds(off, L)], v, mask=m)
        return off + plsc.all_reduce_population_count(m)[0]

      surv[pl.ds(s, L)] = jnp.full((L,), jnp.inf, jnp.float32)
      nv = (s + L - 1) >> 4

      lo0 = lax.bitcast_convert_type(
          bstar.astype(jnp.float32) * jnp.float32(1.0 / NB), jnp.int32)
      hi0 = lax.bitcast_convert_type(
          (bstar + 1).astype(jnp.float32) * jnp.float32(1.0 / NB),
          jnp.int32) - 1

      def bs_cond(c):
        lo, hi = c
        return lo < hi

      def bs_step(c):
        lo, hi = c
        mid = lo + ((hi - lo) >> 1)

        def bcnt(i, a):
          vi = plsc.bitcast(surv[pl.ds(i * L, L)], jnp.int32)
          return a + jnp.sum((vi <= mid).astype(jnp.int32))
        cnt = lax.fori_loop(0, nv, bcnt, jnp.int32(0))

        ge = cnt >= kpp
        return jnp.where(ge, lo, mid + 1), jnp.where(ge, mid, hi)

      tpat, _ = lax.while_loop(bs_cond, bs_step, (lo0, hi0))
      plsc.store_scatter(
          thr, [jnp.full((L,), row_slot, jnp.int32)],
          jnp.full((L,), tpat, jnp.int32), mask=lane == 0)

    def chunk_pair(cp, _):
      ch0 = cp * 2
      wait(ch0, 0, sem0)
      start(ch0 + 1, 1, sem1)

      def rows_a(rr, _):
        row_body(buf.at[0], rr, ch0 * C + rr)
        return 0
      lax.fori_loop(0, C, rows_a, 0)

      wait(ch0 + 1, 1, sem1)

      @pl.when(ch0 + 2 < nch)
      def _():
        start(ch0 + 2, 0, sem0)

      def rows_b(rr, _):
        row_body(buf.at[1], rr, (ch0 + 1) * C + rr)
        return 0
      lax.fori_loop(0, C, rows_b, 0)
      return 0

    start(0, 0, sem0)
    lax.fori_loop(0, nch // 2, chunk_pair, 0)
    pltpu.sync_copy(thr, out_hbm.at[pl.ds(base, rpw)])

  return body(rf)


def _tc_mask(rf, thrf):
  R, G = rf.shape
  blk = 256

  def body(x_ref, t_ref, o_ref):
    o_ref[...] = x_ref[...] >= t_ref[...]

  return pl.pallas_call(
      body,
      grid=(R // blk,),
      in_specs=[
          pl.BlockSpec((blk, G), lambda i: (i, 0)),
          pl.BlockSpec((blk, 1), lambda i: (i, 0)),
      ],
      out_specs=pl.BlockSpec((blk, G), lambda i: (i, 0)),
      out_shape=jax.ShapeDtypeStruct((R, G), jnp.bool_),
  )(rf, thrf)


def _gen_slice(offset, m):
  i = lax.iota(jnp.uint32, m) + jnp.uint32(offset)
  z = jnp.zeros((m,), jnp.uint32)
  b1, b2 = threefry2x32_p.bind(jnp.uint32(0), jnp.uint32(42), z, i)
  fb = ((b1 ^ b2) >> 9) | jnp.uint32(0x3F800000)
  return lax.bitcast_convert_type(fb, jnp.float32) - jnp.float32(1.0)


@jax.jit
def kernel(expression):
  B, N, G = expression.shape
  num_masked = max(1, int(G * EXPRESSION_MASK_RATIO))
  num_masked = min(num_masked, G - MIN_VISIBLE_GENES)
  k1 = G - num_masked
  R = B * N
  Q = 4
  rpc = R // Q
  masks = []
  for q in range(Q):
    rfq = _gen_slice(q * rpc * G, rpc * G).reshape(rpc, G)
    thr_pat = _sc_thresholds(rfq, k1)
    thrf = lax.bitcast_convert_type(thr_pat, jnp.float32).reshape(-1, 1)
    masks.append(_tc_mask(rfq, thrf))
  return jnp.concatenate(masks, axis=0).reshape(B, N, G)
